# Initial kernel scaffold; baseline (speedup 1.0000x reference)
import jax
import jax.numpy as jnp
from jax.experimental import pallas as pl


def kernel(node_feats, edge_feats, edge_index, edge_W1, edge_b1, edge_g1, edge_be1, edge_W2, edge_b2, edge_g2, edge_be2, node_W1, node_b1, node_g1, node_be1, node_W2, node_b2, node_g2, node_be2):
    raise NotImplementedError("write your pallas kernel here")


# speedup vs baseline: 2.1647x; 2.1647x over previous
import functools

import jax
import jax.numpy as jnp
from jax import lax
from jax.experimental import pallas as pl
from jax.experimental.pallas import tpu as pltpu
from jax.experimental.pallas import tpu_sc as plsc

_EPS = 1e-5
_NC = 2
_NS = 16
_NW = _NC * _NS
_CH = 80
_BLK = 2560
_CW = 16


def _proj_body(nf_ref, ws_ref, wd_ref, ps_ref, pd_ref):
    nf = nf_ref[...]
    ps_ref[...] = jnp.dot(nf, ws_ref[...], preferred_element_type=jnp.float32)
    pd_ref[...] = jnp.dot(nf, wd_ref[...], preferred_element_type=jnp.float32)


def _tc_proj(nf, ws, wd):
    n, d = nf.shape
    return pl.pallas_call(
        _proj_body,
        out_shape=(jax.ShapeDtypeStruct((n, d), jnp.float32),
                   jax.ShapeDtypeStruct((n, d), jnp.float32)),
    )(nf, ws, wd)


def _accum_stats(h, s_ref, q_ref):
    ps = jnp.sum(h, axis=0, keepdims=True)
    pq = jnp.sum(h * h, axis=0, keepdims=True)

    @pl.when(pl.program_id(0) == 0)
    def _():
        s_ref[...] = ps
        q_ref[...] = pq

    @pl.when(pl.program_id(0) != 0)
    def _():
        s_ref[...] += ps
        q_ref[...] += pq


def _passa_body(gs_ref, gd_ref, ef_ref, w_ref, b_ref, h1_ref, s_ref, q_ref):
    h = jnp.dot(ef_ref[...], w_ref[...], preferred_element_type=jnp.float32)
    h = h + gs_ref[...] + gd_ref[...] + b_ref[...]
    h1_ref[...] = h
    _accum_stats(h, s_ref, q_ref)


def _bn_affine(s, q, g, be, n):
    mu = s * (1.0 / n)
    var = q * (1.0 / n) - mu * mu
    inv = lax.rsqrt(var + _EPS) * g
    return inv, be - mu * inv


def _silu(x):
    return x * jax.nn.sigmoid(x)


def _passb_body(h1_ref, s1_ref, q1_ref, g1_ref, be1_ref, w2_ref, b2_ref,
                h2_ref, s_ref, q_ref, *, n_rows):
    scale, shift = _bn_affine(s1_ref[...], q1_ref[...], g1_ref[...],
                              be1_ref[...], n_rows)
    a = _silu(h1_ref[...] * scale + shift)
    h2 = jnp.dot(a, w2_ref[...], preferred_element_type=jnp.float32)
    h2 = h2 + b2_ref[...]
    h2_ref[...] = h2
    _accum_stats(h2, s_ref, q_ref)


def _passc_body(h2_ref, ef_ref, s2_ref, q2_ref, g2_ref, be2_ref, ue_ref,
                *, n_rows):
    scale, shift = _bn_affine(s2_ref[...], q2_ref[...], g2_ref[...],
                              be2_ref[...], n_rows)
    ue_ref[...] = _silu(h2_ref[...] * scale + shift + ef_ref[...])


def _edge_grid_call(body, e, d, row_ins, small_ins, n_small_out):
    g = e // _BLK
    bs_rows = pl.BlockSpec((_BLK, d), lambda i: (i, 0))
    bs_vec = pl.BlockSpec((1, d), lambda i: (0, 0))
    bs_mat = pl.BlockSpec((d, d), lambda i: (0, 0))
    in_specs = [bs_rows] * len(row_ins)
    for x in small_ins:
        in_specs.append(bs_mat if x.shape[0] == d else bs_vec)
    out_shape = [jax.ShapeDtypeStruct((e, d), jnp.float32)]
    out_specs = [bs_rows]
    for _ in range(n_small_out):
        out_shape.append(jax.ShapeDtypeStruct((1, d), jnp.float32))
        out_specs.append(bs_vec)
    return pl.pallas_call(
        body,
        grid=(g,),
        in_specs=in_specs,
        out_specs=out_specs,
        out_shape=out_shape,
    )(*row_ins, *small_ins)


def _node_body(nf_ref, agg_ref, cnt_ref,
               w1a_ref, w1b_ref, b1_ref, g1_ref, be1_ref,
               w2_ref, b2_ref, g2_ref, be2_ref, un_ref, *, n_rows):
    nf = nf_ref[...]
    n = nf.shape[0]
    deg = jnp.sum(cnt_ref[...], axis=1, keepdims=True)[0:n]
    agg = jnp.concatenate([agg_ref[0], agg_ref[1]], axis=0)[0:n]
    hg = agg / jnp.maximum(deg, 1.0)
    h = (jnp.dot(nf, w1a_ref[...], preferred_element_type=jnp.float32)
         + jnp.dot(hg, w1b_ref[...], preferred_element_type=jnp.float32)
         + b1_ref[...])
    mu = jnp.mean(h, axis=0, keepdims=True)
    var = jnp.mean(h * h, axis=0, keepdims=True) - mu * mu
    h = (h - mu) * lax.rsqrt(var + _EPS) * g1_ref[...] + be1_ref[...]
    h = _silu(h)
    h = jnp.dot(h, w2_ref[...], preferred_element_type=jnp.float32) + b2_ref[...]
    mu = jnp.mean(h, axis=0, keepdims=True)
    var = jnp.mean(h * h, axis=0, keepdims=True) - mu * mu
    h = (h - mu) * lax.rsqrt(var + _EPS) * g2_ref[...] + be2_ref[...]
    un_ref[...] = _silu(h + nf)


def _tc_node(nf, agg, cnt, w1a, w1b, b1, g1, be1, w2, b2, g2, be2):
    n, d = nf.shape
    return pl.pallas_call(
        functools.partial(_node_body, n_rows=float(n)),
        out_shape=jax.ShapeDtypeStruct((n, d), jnp.float32),
    )(nf, agg, cnt, w1a, w1b, b1, g1, be1, w2, b2, g2, be2)


def _sc_gather(ps, pd, src, dst):
    n, d = ps.shape
    e = src.shape[0]
    per_w = e // _NW
    nch = per_w // _CH
    mesh = plsc.VectorSubcoreMesh(core_axis_name="c", subcore_axis_name="s")

    @functools.partial(
        pl.kernel,
        out_type=(jax.ShapeDtypeStruct((e, d), jnp.float32),
                  jax.ShapeDtypeStruct((e, d), jnp.float32)),
        mesh=mesh,
        scratch_types=[
            pltpu.VMEM((_CH,), jnp.int32),
            pltpu.VMEM((_CH,), jnp.int32),
            pltpu.VMEM((_CH, d), jnp.float32),
            pltpu.VMEM((_CH, d), jnp.float32),
            pltpu.SemaphoreType.DMA,
            pltpu.SemaphoreType.DMA,
        ],
    )
    def k(ps_hbm, pd_hbm, src_hbm, dst_hbm, gs_hbm, gd_hbm,
          idx_s, idx_d, row_s, row_d, sem_s, sem_d):
        wid = lax.axis_index("s") * _NC + lax.axis_index("c")
        base = pl.multiple_of(wid * per_w, 8)

        def body(i, carry):
            off = pl.multiple_of(base + i * _CH, 8)
            pltpu.sync_copy(src_hbm.at[pl.ds(off, _CH)], idx_s)
            pltpu.sync_copy(dst_hbm.at[pl.ds(off, _CH)], idx_d)
            cp_s = pltpu.async_copy(ps_hbm.at[idx_s], row_s, sem_s)
            cp_d = pltpu.async_copy(pd_hbm.at[idx_d], row_d, sem_d)
            cp_s.wait()
            cp_d.wait()
            pltpu.sync_copy(row_s, gs_hbm.at[pl.ds(off, _CH)])
            pltpu.sync_copy(row_d, gd_hbm.at[pl.ds(off, _CH)])
            return carry

        lax.fori_loop(0, nch, body, 0)

    return k(ps, pd, src, dst)


def _sc_scatter(ue, dst, z_nd, half):
    e, d = ue.shape
    per_s = e // _NS
    nch = per_s // _CH
    rps = half // _NS
    mesh = plsc.VectorSubcoreMesh(core_axis_name="c", subcore_axis_name="s")

    nb = _NC * half

    @functools.partial(
        pl.kernel,
        out_type=(jax.ShapeDtypeStruct((_NC, half, d), jnp.float32),
                  jax.ShapeDtypeStruct((_NS * nb,), jnp.float32)),
        mesh=mesh,
        compiler_params=pltpu.CompilerParams(needs_layout_passes=False),
        scratch_types=[
            pltpu.VMEM((_CH,), jnp.int32),
            pltpu.VMEM((_CH, d), jnp.float32),
            pltpu.VMEM((nb,), jnp.float32),
            pltpu.VMEM_SHARED((half + 8, d), jnp.float32),
        ],
    )
    def k(ue_hbm, dst_hbm, znd_hbm, z1_hbm, agg_hbm, cnt_hbm,
          idx_v, rows_v, hist_v, s_agg):
        cid = lax.axis_index("c")
        sid = lax.axis_index("s")
        base = pl.multiple_of(sid * per_s, 8)
        r0 = pl.multiple_of(sid * rps, 8)
        lo = cid * half

        pltpu.sync_copy(znd_hbm, rows_v)
        for t in range(rps // _CH):
            pltpu.sync_copy(rows_v, s_agg.at[pl.ds(r0 + t * _CH, _CH)])
        pltpu.sync_copy(z1_hbm, hist_v)
        plsc.subcore_barrier()

        def body(i, carry):
            off = pl.multiple_of(base + i * _CH, 8)
            pltpu.sync_copy(dst_hbm.at[pl.ds(off, _CH)], idx_v)
            pltpu.sync_copy(ue_hbm.at[pl.ds(off, _CH)], rows_v)
            for j in range(_CH // 16):
                g = idx_v[pl.ds(j * 16, 16)]

                @pl.when(cid == 0)
                def _():
                    plsc.addupdate_scatter(hist_v, [g],
                                           jnp.ones((16,), jnp.float32))
                v = g - lo
                ok = (v >= 0) & (v < half)
                idx_v[pl.ds(j * 16, 16)] = jnp.where(ok, v, half)
            pltpu.sync_copy(rows_v, s_agg.at[idx_v], add=True)
            return carry

        lax.fori_loop(0, nch, body, 0)
        plsc.subcore_barrier()

        for t in range(rps // _CH):
            pltpu.sync_copy(s_agg.at[pl.ds(r0 + t * _CH, _CH)], rows_v)
            pltpu.sync_copy(rows_v, agg_hbm.at[cid, pl.ds(r0 + t * _CH, _CH)])

        @pl.when(cid == 0)
        def _():
            pltpu.sync_copy(hist_v, cnt_hbm.at[pl.ds(sid * nb, nb)])

    return k(ue, dst, z_nd, jnp.zeros((nb,), jnp.float32))


def kernel(node_feats, edge_feats, edge_index,
           edge_W1, edge_b1, edge_g1, edge_be1,
           edge_W2, edge_b2, edge_g2, edge_be2,
           node_W1, node_b1, node_g1, node_be1,
           node_W2, node_b2, node_g2, node_be2):
    n, d = node_feats.shape
    e = edge_feats.shape[0]
    src = edge_index[0]
    dst = edge_index[1]

    row = lambda v: v.reshape(1, d)

    ps, pd = _tc_proj(node_feats, edge_W1[0:d], edge_W1[d:2 * d])
    gs, gd = _sc_gather(ps, pd, src, dst)
    h1, s1, q1 = _edge_grid_call(
        _passa_body, e, d, [gs, gd, edge_feats],
        [edge_W1[2 * d:3 * d], row(edge_b1)], 2)
    h2, s2, q2 = _edge_grid_call(
        functools.partial(_passb_body, n_rows=float(e)), e, d, [h1],
        [s1, q1, row(edge_g1), row(edge_be1), edge_W2, row(edge_b2)], 2)
    (ue,) = _edge_grid_call(
        functools.partial(_passc_body, n_rows=float(e)), e, d,
        [h2, edge_feats], [s2, q2, row(edge_g2), row(edge_be2)], 0)

    half = -(-n // (_NC * _NS * 8)) * (_NS * 8)
    z_nd = jnp.zeros((_CH, d), jnp.float32)
    agg, cnt1d = _sc_scatter(ue, dst, z_nd, half)
    cnt = cnt1d.reshape(_NS, _NC * half).T

    un = _tc_node(node_feats, agg, cnt,
                  node_W1[0:d], node_W1[d:2 * d], row(node_b1),
                  row(node_g1), row(node_be1),
                  node_W2, row(node_b2), row(node_g2), row(node_be2))
    return (un, ue)


# speedup vs baseline: 2.7095x; 1.2517x over previous
import functools

import jax
import jax.numpy as jnp
from jax import lax
from jax.experimental import pallas as pl
from jax.experimental.pallas import tpu as pltpu
from jax.experimental.pallas import tpu_sc as plsc

_EPS = 1e-5
_NC = 2
_NS = 16
_NW = _NC * _NS
_CH = 80
_BLK = 2560
_CW = 16


def _proj_body(nf_ref, ws_ref, wd_ref, ps_ref, pd_ref):
    nf = nf_ref[...]
    ps_ref[...] = jnp.dot(nf, ws_ref[...], preferred_element_type=jnp.float32)
    pd_ref[...] = jnp.dot(nf, wd_ref[...], preferred_element_type=jnp.float32)


def _tc_proj(nf, ws, wd):
    n, d = nf.shape
    return pl.pallas_call(
        _proj_body,
        out_shape=(jax.ShapeDtypeStruct((n, d), jnp.float32),
                   jax.ShapeDtypeStruct((n, d), jnp.float32)),
    )(nf, ws, wd)


def _accum_stats(h, s_ref, q_ref):
    ps = jnp.sum(h, axis=0, keepdims=True)
    pq = jnp.sum(h * h, axis=0, keepdims=True)

    @pl.when(pl.program_id(0) == 0)
    def _():
        s_ref[...] = ps
        q_ref[...] = pq

    @pl.when(pl.program_id(0) != 0)
    def _():
        s_ref[...] += ps
        q_ref[...] += pq


def _passa_body(gs_ref, gd_ref, ef_ref, w_ref, b_ref, h1_ref, s_ref, q_ref):
    h = jnp.dot(ef_ref[...], w_ref[...], preferred_element_type=jnp.float32)
    h = h + gs_ref[...] + gd_ref[...] + b_ref[...]
    h1_ref[...] = h
    _accum_stats(h, s_ref, q_ref)


def _bn_affine(s, q, g, be, n):
    mu = s * (1.0 / n)
    var = q * (1.0 / n) - mu * mu
    inv = lax.rsqrt(var + _EPS) * g
    return inv, be - mu * inv


def _silu(x):
    return x * jax.nn.sigmoid(x)


def _passb_body(h1_ref, s1_ref, q1_ref, g1_ref, be1_ref, w2_ref, b2_ref,
                h2_ref, s_ref, q_ref, *, n_rows):
    scale, shift = _bn_affine(s1_ref[...], q1_ref[...], g1_ref[...],
                              be1_ref[...], n_rows)
    a = _silu(h1_ref[...] * scale + shift)
    h2 = jnp.dot(a, w2_ref[...], preferred_element_type=jnp.float32)
    h2 = h2 + b2_ref[...]
    h2_ref[...] = h2
    _accum_stats(h2, s_ref, q_ref)


def _passc_body(h2_ref, ef_ref, s2_ref, q2_ref, g2_ref, be2_ref, ue_ref,
                *, n_rows):
    scale, shift = _bn_affine(s2_ref[...], q2_ref[...], g2_ref[...],
                              be2_ref[...], n_rows)
    ue_ref[...] = _silu(h2_ref[...] * scale + shift + ef_ref[...])


def _edge_grid_call(body, e, d, row_ins, small_ins, n_small_out):
    g = e // _BLK
    bs_rows = pl.BlockSpec((_BLK, d), lambda i: (i, 0))
    bs_vec = pl.BlockSpec((1, d), lambda i: (0, 0))
    bs_mat = pl.BlockSpec((d, d), lambda i: (0, 0))
    in_specs = [bs_rows] * len(row_ins)
    for x in small_ins:
        in_specs.append(bs_mat if x.shape[0] == d else bs_vec)
    out_shape = [jax.ShapeDtypeStruct((e, d), jnp.float32)]
    out_specs = [bs_rows]
    for _ in range(n_small_out):
        out_shape.append(jax.ShapeDtypeStruct((1, d), jnp.float32))
        out_specs.append(bs_vec)
    return pl.pallas_call(
        body,
        grid=(g,),
        in_specs=in_specs,
        out_specs=out_specs,
        out_shape=out_shape,
    )(*row_ins, *small_ins)


def _node_body(nf_ref, agg_ref, cnt_ref,
               w1a_ref, w1b_ref, b1_ref, g1_ref, be1_ref,
               w2_ref, b2_ref, g2_ref, be2_ref, un_ref, *, n_rows):
    nf = nf_ref[...]
    n = nf.shape[0]
    deg = jnp.sum(cnt_ref[...], axis=1, keepdims=True)[0:n]
    agg = jnp.concatenate([agg_ref[0], agg_ref[1]], axis=0)[0:n]
    hg = agg / jnp.maximum(deg, 1.0)
    h = (jnp.dot(nf, w1a_ref[...], preferred_element_type=jnp.float32)
         + jnp.dot(hg, w1b_ref[...], preferred_element_type=jnp.float32)
         + b1_ref[...])
    mu = jnp.mean(h, axis=0, keepdims=True)
    var = jnp.mean(h * h, axis=0, keepdims=True) - mu * mu
    h = (h - mu) * lax.rsqrt(var + _EPS) * g1_ref[...] + be1_ref[...]
    h = _silu(h)
    h = jnp.dot(h, w2_ref[...], preferred_element_type=jnp.float32) + b2_ref[...]
    mu = jnp.mean(h, axis=0, keepdims=True)
    var = jnp.mean(h * h, axis=0, keepdims=True) - mu * mu
    h = (h - mu) * lax.rsqrt(var + _EPS) * g2_ref[...] + be2_ref[...]
    un_ref[...] = _silu(h + nf)


def _tc_node(nf, agg, cnt, w1a, w1b, b1, g1, be1, w2, b2, g2, be2):
    n, d = nf.shape
    return pl.pallas_call(
        functools.partial(_node_body, n_rows=float(n)),
        out_shape=jax.ShapeDtypeStruct((n, d), jnp.float32),
    )(nf, agg, cnt, w1a, w1b, b1, g1, be1, w2, b2, g2, be2)


def _sc_gather(ps, pd, src, dst):
    n, d = ps.shape
    e = src.shape[0]
    per_w = e // _NW
    nch = per_w // _CH
    mesh = plsc.VectorSubcoreMesh(core_axis_name="c", subcore_axis_name="s")

    @functools.partial(
        pl.kernel,
        out_type=(jax.ShapeDtypeStruct((e, d), jnp.float32),
                  jax.ShapeDtypeStruct((e, d), jnp.float32)),
        mesh=mesh,
        scratch_types=[
            pltpu.VMEM((_CH,), jnp.int32),
            pltpu.VMEM((_CH,), jnp.int32),
            pltpu.VMEM((_CH,), jnp.int32),
            pltpu.VMEM((_CH,), jnp.int32),
            pltpu.VMEM((_CH, d), jnp.float32),
            pltpu.VMEM((_CH, d), jnp.float32),
            pltpu.VMEM((_CH, d), jnp.float32),
            pltpu.VMEM((_CH, d), jnp.float32),
            pltpu.SemaphoreType.DMA,
            pltpu.SemaphoreType.DMA,
            pltpu.SemaphoreType.DMA,
            pltpu.SemaphoreType.DMA,
            pltpu.SemaphoreType.DMA,
            pltpu.SemaphoreType.DMA,
        ],
    )
    def k(ps_hbm, pd_hbm, src_hbm, dst_hbm, gs_hbm, gd_hbm,
          idx_s0, idx_d0, idx_s1, idx_d1, row_s0, row_d0, row_s1, row_d1,
          sem_i0, sem_i1, sem_g0, sem_g1, sem_w0, sem_w1):
        wid = lax.axis_index("s") * _NC + lax.axis_index("c")
        base = pl.multiple_of(wid * per_w, 8)

        def chunk_pair(off0, off1, carry):
            a0 = pltpu.async_copy(src_hbm.at[pl.ds(off0, _CH)], idx_s0, sem_i0)
            b0 = pltpu.async_copy(dst_hbm.at[pl.ds(off0, _CH)], idx_d0, sem_i0)
            a1 = pltpu.async_copy(src_hbm.at[pl.ds(off1, _CH)], idx_s1, sem_i1)
            b1 = pltpu.async_copy(dst_hbm.at[pl.ds(off1, _CH)], idx_d1, sem_i1)
            a0.wait()
            b0.wait()
            g0 = pltpu.async_copy(ps_hbm.at[idx_s0], row_s0, sem_g0)
            h0 = pltpu.async_copy(pd_hbm.at[idx_d0], row_d0, sem_g0)
            a1.wait()
            b1.wait()
            g1 = pltpu.async_copy(ps_hbm.at[idx_s1], row_s1, sem_g1)
            h1 = pltpu.async_copy(pd_hbm.at[idx_d1], row_d1, sem_g1)
            g0.wait()
            h0.wait()
            w0 = pltpu.async_copy(row_s0, gs_hbm.at[pl.ds(off0, _CH)], sem_w0)
            x0 = pltpu.async_copy(row_d0, gd_hbm.at[pl.ds(off0, _CH)], sem_w0)
            g1.wait()
            h1.wait()
            w1 = pltpu.async_copy(row_s1, gs_hbm.at[pl.ds(off1, _CH)], sem_w1)
            x1 = pltpu.async_copy(row_d1, gd_hbm.at[pl.ds(off1, _CH)], sem_w1)
            w0.wait()
            x0.wait()
            w1.wait()
            x1.wait()
            return carry

        def body(t, carry):
            off0 = pl.multiple_of(base + (2 * t) * _CH, 8)
            off1 = pl.multiple_of(base + (2 * t + 1) * _CH, 8)
            return chunk_pair(off0, off1, carry)

        lax.fori_loop(0, nch // 2, body, 0)
        if nch % 2:
            off = pl.multiple_of(base + (nch - 1) * _CH, 8)
            pltpu.sync_copy(src_hbm.at[pl.ds(off, _CH)], idx_s0)
            pltpu.sync_copy(dst_hbm.at[pl.ds(off, _CH)], idx_d0)
            cs = pltpu.async_copy(ps_hbm.at[idx_s0], row_s0, sem_g0)
            cd = pltpu.async_copy(pd_hbm.at[idx_d0], row_d0, sem_g1)
            cs.wait()
            cd.wait()
            pltpu.sync_copy(row_s0, gs_hbm.at[pl.ds(off, _CH)])
            pltpu.sync_copy(row_d0, gd_hbm.at[pl.ds(off, _CH)])

    return k(ps, pd, src, dst)


def _sc_scatter(ue, dst, z_nd, half):
    e, d = ue.shape
    per_s = e // _NS
    nch = per_s // _CH
    rps = half // _NS
    mesh = plsc.VectorSubcoreMesh(core_axis_name="c", subcore_axis_name="s")

    nb = _NC * half

    @functools.partial(
        pl.kernel,
        out_type=(jax.ShapeDtypeStruct((_NC, half, d), jnp.float32),
                  jax.ShapeDtypeStruct((_NS * nb,), jnp.float32)),
        mesh=mesh,
        compiler_params=pltpu.CompilerParams(needs_layout_passes=False),
        scratch_types=[
            pltpu.VMEM((_CH,), jnp.int32),
            pltpu.VMEM((_CH,), jnp.int32),
            pltpu.VMEM((_CH, d), jnp.float32),
            pltpu.VMEM((_CH, d), jnp.float32),
            pltpu.VMEM((nb,), jnp.float32),
            pltpu.SemaphoreType.DMA,
            pltpu.SemaphoreType.DMA,
            pltpu.SemaphoreType.DMA,
            pltpu.SemaphoreType.DMA,
            pltpu.VMEM_SHARED((half + 8, d), jnp.float32),
        ],
    )
    def k(ue_hbm, dst_hbm, znd_hbm, z1_hbm, agg_hbm, cnt_hbm,
          idx_v, idx_v1, rows_v, rows_v1, hist_v,
          sem_i0, sem_i1, sem_r0, sem_r1, s_agg):
        cid = lax.axis_index("c")
        sid = lax.axis_index("s")
        base = pl.multiple_of(sid * per_s, 8)
        r0 = pl.multiple_of(sid * rps, 8)
        lo = cid * half

        pltpu.sync_copy(znd_hbm, rows_v)
        for t in range(rps // _CH):
            pltpu.sync_copy(rows_v, s_agg.at[pl.ds(r0 + t * _CH, _CH)])
        pltpu.sync_copy(z1_hbm, hist_v)
        plsc.subcore_barrier()

        def hist_and_remap(ix):
            for j in range(_CH // 16):
                g = ix[pl.ds(j * 16, 16)]

                @pl.when(cid == 0)
                def _():
                    plsc.addupdate_scatter(hist_v, [g],
                                           jnp.ones((16,), jnp.float32))
                v = g - lo
                ok = (v >= 0) & (v < half)
                ix[pl.ds(j * 16, 16)] = jnp.where(ok, v, half)

        def body(t, carry):
            off0 = pl.multiple_of(base + (2 * t) * _CH, 8)
            off1 = pl.multiple_of(base + (2 * t + 1) * _CH, 8)
            i0 = pltpu.async_copy(dst_hbm.at[pl.ds(off0, _CH)], idx_v, sem_i0)
            r0 = pltpu.async_copy(ue_hbm.at[pl.ds(off0, _CH)], rows_v, sem_r0)
            i1 = pltpu.async_copy(dst_hbm.at[pl.ds(off1, _CH)], idx_v1, sem_i1)
            r1 = pltpu.async_copy(ue_hbm.at[pl.ds(off1, _CH)], rows_v1, sem_r1)
            i0.wait()
            hist_and_remap(idx_v)
            r0.wait()
            pltpu.sync_copy(rows_v, s_agg.at[idx_v], add=True)
            i1.wait()
            hist_and_remap(idx_v1)
            r1.wait()
            pltpu.sync_copy(rows_v1, s_agg.at[idx_v1], add=True)
            return carry

        lax.fori_loop(0, nch // 2, body, 0)
        if nch % 2:
            off = pl.multiple_of(base + (nch - 1) * _CH, 8)
            pltpu.sync_copy(dst_hbm.at[pl.ds(off, _CH)], idx_v)
            pltpu.sync_copy(ue_hbm.at[pl.ds(off, _CH)], rows_v)
            hist_and_remap(idx_v)
            pltpu.sync_copy(rows_v, s_agg.at[idx_v], add=True)
        plsc.subcore_barrier()

        for t in range(rps // _CH):
            pltpu.sync_copy(s_agg.at[pl.ds(r0 + t * _CH, _CH)], rows_v)
            pltpu.sync_copy(rows_v, agg_hbm.at[cid, pl.ds(r0 + t * _CH, _CH)])

        @pl.when(cid == 0)
        def _():
            pltpu.sync_copy(hist_v, cnt_hbm.at[pl.ds(sid * nb, nb)])

    return k(ue, dst, z_nd, jnp.zeros((nb,), jnp.float32))


def kernel(node_feats, edge_feats, edge_index,
           edge_W1, edge_b1, edge_g1, edge_be1,
           edge_W2, edge_b2, edge_g2, edge_be2,
           node_W1, node_b1, node_g1, node_be1,
           node_W2, node_b2, node_g2, node_be2):
    n, d = node_feats.shape
    e = edge_feats.shape[0]
    src = edge_index[0]
    dst = edge_index[1]

    row = lambda v: v.reshape(1, d)

    ps, pd = _tc_proj(node_feats, edge_W1[0:d], edge_W1[d:2 * d])
    gs, gd = _sc_gather(ps, pd, src, dst)
    h1, s1, q1 = _edge_grid_call(
        _passa_body, e, d, [gs, gd, edge_feats],
        [edge_W1[2 * d:3 * d], row(edge_b1)], 2)
    h2, s2, q2 = _edge_grid_call(
        functools.partial(_passb_body, n_rows=float(e)), e, d, [h1],
        [s1, q1, row(edge_g1), row(edge_be1), edge_W2, row(edge_b2)], 2)
    (ue,) = _edge_grid_call(
        functools.partial(_passc_body, n_rows=float(e)), e, d,
        [h2, edge_feats], [s2, q2, row(edge_g2), row(edge_be2)], 0)

    half = -(-n // (_NC * _NS * 8)) * (_NS * 8)
    z_nd = jnp.zeros((_CH, d), jnp.float32)
    agg, cnt1d = _sc_scatter(ue, dst, z_nd, half)
    cnt = cnt1d.reshape(_NS, _NC * half).T

    un = _tc_node(node_feats, agg, cnt,
                  node_W1[0:d], node_W1[d:2 * d], row(node_b1),
                  row(node_g1), row(node_be1),
                  node_W2, row(node_b2), row(node_g2), row(node_be2))
    return (un, ue)


# speedup vs baseline: 2.8017x; 1.0340x over previous
import functools

import jax
import jax.numpy as jnp
from jax import lax
from jax.experimental import pallas as pl
from jax.experimental.pallas import tpu as pltpu
from jax.experimental.pallas import tpu_sc as plsc

_EPS = 1e-5
_NC = 2
_NS = 16
_NW = _NC * _NS
_CH = 80
_BLK = 2560
_CW = 16


def _proj_body(nf_ref, ws_ref, wd_ref, ps_ref, pd_ref):
    nf = nf_ref[...]
    ps_ref[...] = jnp.dot(nf, ws_ref[...], preferred_element_type=jnp.float32)
    pd_ref[...] = jnp.dot(nf, wd_ref[...], preferred_element_type=jnp.float32)


def _tc_proj(nf, ws, wd):
    n, d = nf.shape
    return pl.pallas_call(
        _proj_body,
        out_shape=(jax.ShapeDtypeStruct((n, d), jnp.float32),
                   jax.ShapeDtypeStruct((n, d), jnp.float32)),
    )(nf, ws, wd)


def _accum_stats(h, s_ref, q_ref):
    ps = jnp.sum(h, axis=0, keepdims=True)
    pq = jnp.sum(h * h, axis=0, keepdims=True)

    @pl.when(pl.program_id(0) == 0)
    def _():
        s_ref[...] = ps
        q_ref[...] = pq

    @pl.when(pl.program_id(0) != 0)
    def _():
        s_ref[...] += ps
        q_ref[...] += pq


def _passa_body(gs_ref, gd_ref, ef_ref, w_ref, b_ref, h1_ref, s_ref, q_ref):
    h = jnp.dot(ef_ref[...], w_ref[...], preferred_element_type=jnp.float32)
    h = h + gs_ref[...] + gd_ref[...] + b_ref[...]
    h1_ref[...] = h
    _accum_stats(h, s_ref, q_ref)


def _bn_affine(s, q, g, be, n):
    mu = s * (1.0 / n)
    var = q * (1.0 / n) - mu * mu
    inv = lax.rsqrt(var + _EPS) * g
    return inv, be - mu * inv


def _silu(x):
    return x * jax.nn.sigmoid(x)


def _passb_body(h1_ref, s1_ref, q1_ref, g1_ref, be1_ref, w2_ref, b2_ref,
                h2_ref, s_ref, q_ref, *, n_rows):
    scale, shift = _bn_affine(s1_ref[...], q1_ref[...], g1_ref[...],
                              be1_ref[...], n_rows)
    a = _silu(h1_ref[...] * scale + shift)
    h2 = jnp.dot(a, w2_ref[...], preferred_element_type=jnp.float32)
    h2 = h2 + b2_ref[...]
    h2_ref[...] = h2
    _accum_stats(h2, s_ref, q_ref)


def _passc_body(h2_ref, ef_ref, s2_ref, q2_ref, g2_ref, be2_ref, ue_ref,
                *, n_rows):
    scale, shift = _bn_affine(s2_ref[...], q2_ref[...], g2_ref[...],
                              be2_ref[...], n_rows)
    ue_ref[...] = _silu(h2_ref[...] * scale + shift + ef_ref[...])


def _edge_grid_call(body, e, d, row_ins, small_ins, n_small_out):
    g = e // _BLK
    bs_rows = pl.BlockSpec((_BLK, d), lambda i: (i, 0))
    bs_vec = pl.BlockSpec((1, d), lambda i: (0, 0))
    bs_mat = pl.BlockSpec((d, d), lambda i: (0, 0))
    in_specs = [bs_rows] * len(row_ins)
    for x in small_ins:
        in_specs.append(bs_mat if x.shape[0] == d else bs_vec)
    out_shape = [jax.ShapeDtypeStruct((e, d), jnp.float32)]
    out_specs = [bs_rows]
    for _ in range(n_small_out):
        out_shape.append(jax.ShapeDtypeStruct((1, d), jnp.float32))
        out_specs.append(bs_vec)
    return pl.pallas_call(
        body,
        grid=(g,),
        in_specs=in_specs,
        out_specs=out_specs,
        out_shape=out_shape,
    )(*row_ins, *small_ins)


def _node_body(nf_ref, agg_ref, cnt_ref,
               w1a_ref, w1b_ref, b1_ref, g1_ref, be1_ref,
               w2_ref, b2_ref, g2_ref, be2_ref, un_ref, *, n_rows):
    nf = nf_ref[...]
    n = nf.shape[0]
    deg = jnp.sum(cnt_ref[...], axis=1, keepdims=True)[0:n]
    agg = jnp.concatenate([agg_ref[0], agg_ref[1]], axis=0)[0:n]
    hg = agg / jnp.maximum(deg, 1.0)
    h = (jnp.dot(nf, w1a_ref[...], preferred_element_type=jnp.float32)
         + jnp.dot(hg, w1b_ref[...], preferred_element_type=jnp.float32)
         + b1_ref[...])
    mu = jnp.mean(h, axis=0, keepdims=True)
    var = jnp.mean(h * h, axis=0, keepdims=True) - mu * mu
    h = (h - mu) * lax.rsqrt(var + _EPS) * g1_ref[...] + be1_ref[...]
    h = _silu(h)
    h = jnp.dot(h, w2_ref[...], preferred_element_type=jnp.float32) + b2_ref[...]
    mu = jnp.mean(h, axis=0, keepdims=True)
    var = jnp.mean(h * h, axis=0, keepdims=True) - mu * mu
    h = (h - mu) * lax.rsqrt(var + _EPS) * g2_ref[...] + be2_ref[...]
    un_ref[...] = _silu(h + nf)


def _tc_node(nf, agg, cnt, w1a, w1b, b1, g1, be1, w2, b2, g2, be2):
    n, d = nf.shape
    return pl.pallas_call(
        functools.partial(_node_body, n_rows=float(n)),
        out_shape=jax.ShapeDtypeStruct((n, d), jnp.float32),
    )(nf, agg, cnt, w1a, w1b, b1, g1, be1, w2, b2, g2, be2)


def _sc_gather(ps, pd, src, dst):
    n, d = ps.shape
    e = src.shape[0]
    per_w = e // _NW
    nch = per_w // _CH
    mesh = plsc.VectorSubcoreMesh(core_axis_name="c", subcore_axis_name="s")

    nq = 4

    @functools.partial(
        pl.kernel,
        out_type=(jax.ShapeDtypeStruct((e, d), jnp.float32),
                  jax.ShapeDtypeStruct((e, d), jnp.float32)),
        mesh=mesh,
        scratch_types=(
            [pltpu.VMEM((_CH,), jnp.int32)] * (2 * nq)
            + [pltpu.VMEM((_CH, d), jnp.float32)] * (2 * nq)
            + [pltpu.SemaphoreType.DMA] * (3 * nq)
        ),
    )
    def k(ps_hbm, pd_hbm, src_hbm, dst_hbm, gs_hbm, gd_hbm, *bufs):
        idx_s = bufs[0:nq]
        idx_d = bufs[nq:2 * nq]
        row_s = bufs[2 * nq:3 * nq]
        row_d = bufs[3 * nq:4 * nq]
        sem_i = bufs[4 * nq:5 * nq]
        sem_g = bufs[5 * nq:6 * nq]
        sem_w = bufs[6 * nq:7 * nq]
        wid = lax.axis_index("s") * _NC + lax.axis_index("c")
        base = pl.multiple_of(wid * per_w, 8)

        def body(t, carry):
            offs = [pl.multiple_of(base + (nq * t + q) * _CH, 8)
                    for q in range(nq)]
            ii = [(pltpu.async_copy(src_hbm.at[pl.ds(offs[q], _CH)],
                                    idx_s[q], sem_i[q]),
                   pltpu.async_copy(dst_hbm.at[pl.ds(offs[q], _CH)],
                                    idx_d[q], sem_i[q]))
                  for q in range(nq)]
            gg = []
            for q in range(nq):
                ii[q][0].wait()
                ii[q][1].wait()
                gg.append((pltpu.async_copy(ps_hbm.at[idx_s[q]], row_s[q],
                                            sem_g[q]),
                           pltpu.async_copy(pd_hbm.at[idx_d[q]], row_d[q],
                                            sem_g[q])))
            ww = []
            for q in range(nq):
                gg[q][0].wait()
                gg[q][1].wait()
                ww.append((pltpu.async_copy(row_s[q],
                                            gs_hbm.at[pl.ds(offs[q], _CH)],
                                            sem_w[q]),
                           pltpu.async_copy(row_d[q],
                                            gd_hbm.at[pl.ds(offs[q], _CH)],
                                            sem_w[q])))
            for q in range(nq):
                ww[q][0].wait()
                ww[q][1].wait()
            return carry

        lax.fori_loop(0, nch // nq, body, 0)
        for r in range(nch - (nch // nq) * nq):
            off = pl.multiple_of(base + ((nch // nq) * nq + r) * _CH, 8)
            pltpu.sync_copy(src_hbm.at[pl.ds(off, _CH)], idx_s[0])
            pltpu.sync_copy(dst_hbm.at[pl.ds(off, _CH)], idx_d[0])
            cs = pltpu.async_copy(ps_hbm.at[idx_s[0]], row_s[0], sem_g[0])
            cd = pltpu.async_copy(pd_hbm.at[idx_d[0]], row_d[0], sem_g[1])
            cs.wait()
            cd.wait()
            pltpu.sync_copy(row_s[0], gs_hbm.at[pl.ds(off, _CH)])
            pltpu.sync_copy(row_d[0], gd_hbm.at[pl.ds(off, _CH)])

    return k(ps, pd, src, dst)


def _sc_scatter(ue, dst, z_nd, half):
    e, d = ue.shape
    per_s = e // _NS
    nch = per_s // _CH
    rps = half // _NS
    mesh = plsc.VectorSubcoreMesh(core_axis_name="c", subcore_axis_name="s")

    nb = _NC * half

    @functools.partial(
        pl.kernel,
        out_type=(jax.ShapeDtypeStruct((_NC, half, d), jnp.float32),
                  jax.ShapeDtypeStruct((_NS * nb,), jnp.float32)),
        mesh=mesh,
        compiler_params=pltpu.CompilerParams(needs_layout_passes=False),
        scratch_types=[
            pltpu.VMEM((_CH,), jnp.int32),
            pltpu.VMEM((_CH,), jnp.int32),
            pltpu.VMEM((_CH, d), jnp.float32),
            pltpu.VMEM((_CH, d), jnp.float32),
            pltpu.VMEM((nb,), jnp.float32),
            pltpu.SemaphoreType.DMA,
            pltpu.SemaphoreType.DMA,
            pltpu.SemaphoreType.DMA,
            pltpu.SemaphoreType.DMA,
            pltpu.VMEM_SHARED((half + 8, d), jnp.float32),
        ],
    )
    def k(ue_hbm, dst_hbm, znd_hbm, z1_hbm, agg_hbm, cnt_hbm,
          idx_v, idx_v1, rows_v, rows_v1, hist_v,
          sem_i0, sem_i1, sem_r0, sem_r1, s_agg):
        cid = lax.axis_index("c")
        sid = lax.axis_index("s")
        base = pl.multiple_of(sid * per_s, 8)
        r0 = pl.multiple_of(sid * rps, 8)
        lo = cid * half

        pltpu.sync_copy(znd_hbm, rows_v)
        for t in range(rps // _CH):
            pltpu.sync_copy(rows_v, s_agg.at[pl.ds(r0 + t * _CH, _CH)])
        pltpu.sync_copy(z1_hbm, hist_v)
        plsc.subcore_barrier()

        def hist_and_remap(ix):
            for j in range(_CH // 16):
                g = ix[pl.ds(j * 16, 16)]

                @pl.when(cid == 0)
                def _():
                    plsc.addupdate_scatter(hist_v, [g],
                                           jnp.ones((16,), jnp.float32))
                v = g - lo
                ok = (v >= 0) & (v < half)
                ix[pl.ds(j * 16, 16)] = jnp.where(ok, v, half)

        def body(t, carry):
            off0 = pl.multiple_of(base + (2 * t) * _CH, 8)
            off1 = pl.multiple_of(base + (2 * t + 1) * _CH, 8)
            i0 = pltpu.async_copy(dst_hbm.at[pl.ds(off0, _CH)], idx_v, sem_i0)
            r0 = pltpu.async_copy(ue_hbm.at[pl.ds(off0, _CH)], rows_v, sem_r0)
            i1 = pltpu.async_copy(dst_hbm.at[pl.ds(off1, _CH)], idx_v1, sem_i1)
            r1 = pltpu.async_copy(ue_hbm.at[pl.ds(off1, _CH)], rows_v1, sem_r1)
            i0.wait()
            hist_and_remap(idx_v)
            r0.wait()
            pltpu.sync_copy(rows_v, s_agg.at[idx_v], add=True)
            i1.wait()
            hist_and_remap(idx_v1)
            r1.wait()
            pltpu.sync_copy(rows_v1, s_agg.at[idx_v1], add=True)
            return carry

        lax.fori_loop(0, nch // 2, body, 0)
        if nch % 2:
            off = pl.multiple_of(base + (nch - 1) * _CH, 8)
            pltpu.sync_copy(dst_hbm.at[pl.ds(off, _CH)], idx_v)
            pltpu.sync_copy(ue_hbm.at[pl.ds(off, _CH)], rows_v)
            hist_and_remap(idx_v)
            pltpu.sync_copy(rows_v, s_agg.at[idx_v], add=True)
        plsc.subcore_barrier()

        for t in range(rps // _CH):
            pltpu.sync_copy(s_agg.at[pl.ds(r0 + t * _CH, _CH)], rows_v)
            pltpu.sync_copy(rows_v, agg_hbm.at[cid, pl.ds(r0 + t * _CH, _CH)])

        @pl.when(cid == 0)
        def _():
            pltpu.sync_copy(hist_v, cnt_hbm.at[pl.ds(sid * nb, nb)])

    return k(ue, dst, z_nd, jnp.zeros((nb,), jnp.float32))


def kernel(node_feats, edge_feats, edge_index,
           edge_W1, edge_b1, edge_g1, edge_be1,
           edge_W2, edge_b2, edge_g2, edge_be2,
           node_W1, node_b1, node_g1, node_be1,
           node_W2, node_b2, node_g2, node_be2):
    n, d = node_feats.shape
    e = edge_feats.shape[0]
    src = edge_index[0]
    dst = edge_index[1]

    row = lambda v: v.reshape(1, d)

    ps, pd = _tc_proj(node_feats, edge_W1[0:d], edge_W1[d:2 * d])
    gs, gd = _sc_gather(ps, pd, src, dst)
    h1, s1, q1 = _edge_grid_call(
        _passa_body, e, d, [gs, gd, edge_feats],
        [edge_W1[2 * d:3 * d], row(edge_b1)], 2)
    h2, s2, q2 = _edge_grid_call(
        functools.partial(_passb_body, n_rows=float(e)), e, d, [h1],
        [s1, q1, row(edge_g1), row(edge_be1), edge_W2, row(edge_b2)], 2)
    (ue,) = _edge_grid_call(
        functools.partial(_passc_body, n_rows=float(e)), e, d,
        [h2, edge_feats], [s2, q2, row(edge_g2), row(edge_be2)], 0)

    half = -(-n // (_NC * _NS * 8)) * (_NS * 8)
    z_nd = jnp.zeros((_CH, d), jnp.float32)
    agg, cnt1d = _sc_scatter(ue, dst, z_nd, half)
    cnt = cnt1d.reshape(_NS, _NC * half).T

    un = _tc_node(node_feats, agg, cnt,
                  node_W1[0:d], node_W1[d:2 * d], row(node_b1),
                  row(node_g1), row(node_be1),
                  node_W2, row(node_b2), row(node_g2), row(node_be2))
    return (un, ue)


# speedup vs baseline: 2.9547x; 1.0546x over previous
import functools

import jax
import jax.numpy as jnp
from jax import lax
from jax.experimental import pallas as pl
from jax.experimental.pallas import tpu as pltpu
from jax.experimental.pallas import tpu_sc as plsc

_EPS = 1e-5
_NC = 2
_NS = 16
_NW = _NC * _NS
_CH = 80
_BLK = 2560
_CW = 16


def _proj_body(nf_ref, ws_ref, wd_ref, ps_ref, pd_ref):
    nf = nf_ref[...]
    ps_ref[...] = jnp.dot(nf, ws_ref[...], preferred_element_type=jnp.float32)
    pd_ref[...] = jnp.dot(nf, wd_ref[...], preferred_element_type=jnp.float32)


def _tc_proj(nf, ws, wd):
    n, d = nf.shape
    return pl.pallas_call(
        _proj_body,
        out_shape=(jax.ShapeDtypeStruct((n, d), jnp.float32),
                   jax.ShapeDtypeStruct((n, d), jnp.float32)),
    )(nf, ws, wd)


def _accum_stats(h, s_ref, q_ref):
    ps = jnp.sum(h, axis=0, keepdims=True)
    pq = jnp.sum(h * h, axis=0, keepdims=True)

    @pl.when(pl.program_id(0) == 0)
    def _():
        s_ref[...] = ps
        q_ref[...] = pq

    @pl.when(pl.program_id(0) != 0)
    def _():
        s_ref[...] += ps
        q_ref[...] += pq


def _passa_body(gs_ref, gd_ref, ef_ref, w_ref, b_ref, h1_ref, s_ref, q_ref):
    h = jnp.dot(ef_ref[...], w_ref[...], preferred_element_type=jnp.float32)
    h = h + gs_ref[...] + gd_ref[...] + b_ref[...]
    h1_ref[...] = h.astype(h1_ref.dtype)
    _accum_stats(h, s_ref, q_ref)


def _bn_affine(s, q, g, be, n):
    mu = s * (1.0 / n)
    var = q * (1.0 / n) - mu * mu
    inv = lax.rsqrt(var + _EPS) * g
    return inv, be - mu * inv


def _silu(x):
    return x * jax.nn.sigmoid(x)


def _passb_body(h1_ref, s1_ref, q1_ref, g1_ref, be1_ref, w2_ref, b2_ref,
                h2_ref, s_ref, q_ref, *, n_rows):
    scale, shift = _bn_affine(s1_ref[...], q1_ref[...], g1_ref[...],
                              be1_ref[...], n_rows)
    a = _silu(h1_ref[...].astype(jnp.float32) * scale + shift)
    h2 = jnp.dot(a, w2_ref[...], preferred_element_type=jnp.float32)
    h2 = h2 + b2_ref[...]
    h2_ref[...] = h2.astype(h2_ref.dtype)
    _accum_stats(h2, s_ref, q_ref)


def _passc_body(h2_ref, ef_ref, s2_ref, q2_ref, g2_ref, be2_ref, ue_ref,
                *, n_rows):
    scale, shift = _bn_affine(s2_ref[...], q2_ref[...], g2_ref[...],
                              be2_ref[...], n_rows)
    ue_ref[...] = _silu(h2_ref[...].astype(jnp.float32) * scale + shift
                        + ef_ref[...])


def _edge_grid_call(body, e, d, row_ins, small_ins, n_small_out,
                    out_dtype=jnp.float32):
    g = e // _BLK
    bs_rows = pl.BlockSpec((_BLK, d), lambda i: (i, 0))
    bs_vec = pl.BlockSpec((1, d), lambda i: (0, 0))
    bs_mat = pl.BlockSpec((d, d), lambda i: (0, 0))
    in_specs = [bs_rows] * len(row_ins)
    for x in small_ins:
        in_specs.append(bs_mat if x.shape[0] == d else bs_vec)
    out_shape = [jax.ShapeDtypeStruct((e, d), out_dtype)]
    out_specs = [bs_rows]
    for _ in range(n_small_out):
        out_shape.append(jax.ShapeDtypeStruct((1, d), jnp.float32))
        out_specs.append(bs_vec)
    return pl.pallas_call(
        body,
        grid=(g,),
        in_specs=in_specs,
        out_specs=out_specs,
        out_shape=out_shape,
    )(*row_ins, *small_ins)


def _node_body(nf_ref, agg_ref, cnt_ref,
               w1a_ref, w1b_ref, b1_ref, g1_ref, be1_ref,
               w2_ref, b2_ref, g2_ref, be2_ref, un_ref, *, n_rows):
    nf = nf_ref[...]
    n = nf.shape[0]
    deg = jnp.sum(cnt_ref[...], axis=1, keepdims=True)[0:n]
    agg = jnp.concatenate([agg_ref[0], agg_ref[1]], axis=0)[0:n]
    hg = agg / jnp.maximum(deg, 1.0)
    h = (jnp.dot(nf, w1a_ref[...], preferred_element_type=jnp.float32)
         + jnp.dot(hg, w1b_ref[...], preferred_element_type=jnp.float32)
         + b1_ref[...])
    mu = jnp.mean(h, axis=0, keepdims=True)
    var = jnp.mean(h * h, axis=0, keepdims=True) - mu * mu
    h = (h - mu) * lax.rsqrt(var + _EPS) * g1_ref[...] + be1_ref[...]
    h = _silu(h)
    h = jnp.dot(h, w2_ref[...], preferred_element_type=jnp.float32) + b2_ref[...]
    mu = jnp.mean(h, axis=0, keepdims=True)
    var = jnp.mean(h * h, axis=0, keepdims=True) - mu * mu
    h = (h - mu) * lax.rsqrt(var + _EPS) * g2_ref[...] + be2_ref[...]
    un_ref[...] = _silu(h + nf)


def _tc_node(nf, agg, cnt, w1a, w1b, b1, g1, be1, w2, b2, g2, be2):
    n, d = nf.shape
    return pl.pallas_call(
        functools.partial(_node_body, n_rows=float(n)),
        out_shape=jax.ShapeDtypeStruct((n, d), jnp.float32),
    )(nf, agg, cnt, w1a, w1b, b1, g1, be1, w2, b2, g2, be2)


def _sc_gather(ps, pd, src, dst):
    n, d = ps.shape
    e = src.shape[0]
    per_w = e // _NW
    nch = per_w // _CH
    mesh = plsc.VectorSubcoreMesh(core_axis_name="c", subcore_axis_name="s")

    nq = 4

    @functools.partial(
        pl.kernel,
        out_type=(jax.ShapeDtypeStruct((e, d), jnp.float32),
                  jax.ShapeDtypeStruct((e, d), jnp.float32)),
        mesh=mesh,
        scratch_types=(
            [pltpu.VMEM((_CH,), jnp.int32)] * (2 * nq)
            + [pltpu.VMEM((_CH, d), jnp.float32)] * (2 * nq)
            + [pltpu.SemaphoreType.DMA] * (3 * nq)
        ),
    )
    def k(ps_hbm, pd_hbm, src_hbm, dst_hbm, gs_hbm, gd_hbm, *bufs):
        idx_s = bufs[0:nq]
        idx_d = bufs[nq:2 * nq]
        row_s = bufs[2 * nq:3 * nq]
        row_d = bufs[3 * nq:4 * nq]
        sem_i = bufs[4 * nq:5 * nq]
        sem_g = bufs[5 * nq:6 * nq]
        sem_w = bufs[6 * nq:7 * nq]
        wid = lax.axis_index("s") * _NC + lax.axis_index("c")
        base = pl.multiple_of(wid * per_w, 8)

        def body(t, carry):
            offs = [pl.multiple_of(base + (nq * t + q) * _CH, 8)
                    for q in range(nq)]
            ii = [(pltpu.async_copy(src_hbm.at[pl.ds(offs[q], _CH)],
                                    idx_s[q], sem_i[q]),
                   pltpu.async_copy(dst_hbm.at[pl.ds(offs[q], _CH)],
                                    idx_d[q], sem_i[q]))
                  for q in range(nq)]
            gg = []
            for q in range(nq):
                ii[q][0].wait()
                ii[q][1].wait()
                gg.append((pltpu.async_copy(ps_hbm.at[idx_s[q]], row_s[q],
                                            sem_g[q]),
                           pltpu.async_copy(pd_hbm.at[idx_d[q]], row_d[q],
                                            sem_g[q])))
            ww = []
            for q in range(nq):
                gg[q][0].wait()
                gg[q][1].wait()
                ww.append((pltpu.async_copy(row_s[q],
                                            gs_hbm.at[pl.ds(offs[q], _CH)],
                                            sem_w[q]),
                           pltpu.async_copy(row_d[q],
                                            gd_hbm.at[pl.ds(offs[q], _CH)],
                                            sem_w[q])))
            for q in range(nq):
                ww[q][0].wait()
                ww[q][1].wait()
            return carry

        lax.fori_loop(0, nch // nq, body, 0)
        for r in range(nch - (nch // nq) * nq):
            off = pl.multiple_of(base + ((nch // nq) * nq + r) * _CH, 8)
            pltpu.sync_copy(src_hbm.at[pl.ds(off, _CH)], idx_s[0])
            pltpu.sync_copy(dst_hbm.at[pl.ds(off, _CH)], idx_d[0])
            cs = pltpu.async_copy(ps_hbm.at[idx_s[0]], row_s[0], sem_g[0])
            cd = pltpu.async_copy(pd_hbm.at[idx_d[0]], row_d[0], sem_g[1])
            cs.wait()
            cd.wait()
            pltpu.sync_copy(row_s[0], gs_hbm.at[pl.ds(off, _CH)])
            pltpu.sync_copy(row_d[0], gd_hbm.at[pl.ds(off, _CH)])

    return k(ps, pd, src, dst)


def _sc_scatter(ue, dst, z_nd, half):
    e, d = ue.shape
    per_s = e // _NS
    nch = per_s // _CH
    rps = half // _NS
    mesh = plsc.VectorSubcoreMesh(core_axis_name="c", subcore_axis_name="s")

    nb = _NC * half

    @functools.partial(
        pl.kernel,
        out_type=(jax.ShapeDtypeStruct((_NC, half, d), jnp.float32),
                  jax.ShapeDtypeStruct((_NS * nb,), jnp.float32)),
        mesh=mesh,
        compiler_params=pltpu.CompilerParams(needs_layout_passes=False),
        scratch_types=[
            pltpu.VMEM((_CH,), jnp.int32),
            pltpu.VMEM((_CH,), jnp.int32),
            pltpu.VMEM((_CH, d), jnp.float32),
            pltpu.VMEM((_CH, d), jnp.float32),
            pltpu.VMEM((nb,), jnp.float32),
            pltpu.SemaphoreType.DMA,
            pltpu.SemaphoreType.DMA,
            pltpu.SemaphoreType.DMA,
            pltpu.SemaphoreType.DMA,
            pltpu.VMEM_SHARED((half + 8, d), jnp.float32),
        ],
    )
    def k(ue_hbm, dst_hbm, znd_hbm, z1_hbm, agg_hbm, cnt_hbm,
          idx_v, idx_v1, rows_v, rows_v1, hist_v,
          sem_i0, sem_i1, sem_r0, sem_r1, s_agg):
        cid = lax.axis_index("c")
        sid = lax.axis_index("s")
        base = pl.multiple_of(sid * per_s, 8)
        r0 = pl.multiple_of(sid * rps, 8)
        lo = cid * half

        pltpu.sync_copy(znd_hbm, rows_v)
        for t in range(rps // _CH):
            pltpu.sync_copy(rows_v, s_agg.at[pl.ds(r0 + t * _CH, _CH)])
        pltpu.sync_copy(z1_hbm, hist_v)
        plsc.subcore_barrier()

        def hist_and_remap(ix):
            for j in range(_CH // 16):
                g = ix[pl.ds(j * 16, 16)]

                @pl.when(cid == 0)
                def _():
                    plsc.addupdate_scatter(hist_v, [g],
                                           jnp.ones((16,), jnp.float32))
                v = g - lo
                ok = (v >= 0) & (v < half)
                ix[pl.ds(j * 16, 16)] = jnp.where(ok, v, half)

        def body(t, carry):
            off0 = pl.multiple_of(base + (2 * t) * _CH, 8)
            off1 = pl.multiple_of(base + (2 * t + 1) * _CH, 8)
            i0 = pltpu.async_copy(dst_hbm.at[pl.ds(off0, _CH)], idx_v, sem_i0)
            r0 = pltpu.async_copy(ue_hbm.at[pl.ds(off0, _CH)], rows_v, sem_r0)
            i1 = pltpu.async_copy(dst_hbm.at[pl.ds(off1, _CH)], idx_v1, sem_i1)
            r1 = pltpu.async_copy(ue_hbm.at[pl.ds(off1, _CH)], rows_v1, sem_r1)
            i0.wait()
            hist_and_remap(idx_v)
            r0.wait()
            pltpu.sync_copy(rows_v, s_agg.at[idx_v], add=True)
            i1.wait()
            hist_and_remap(idx_v1)
            r1.wait()
            pltpu.sync_copy(rows_v1, s_agg.at[idx_v1], add=True)
            return carry

        lax.fori_loop(0, nch // 2, body, 0)
        if nch % 2:
            off = pl.multiple_of(base + (nch - 1) * _CH, 8)
            pltpu.sync_copy(dst_hbm.at[pl.ds(off, _CH)], idx_v)
            pltpu.sync_copy(ue_hbm.at[pl.ds(off, _CH)], rows_v)
            hist_and_remap(idx_v)
            pltpu.sync_copy(rows_v, s_agg.at[idx_v], add=True)
        plsc.subcore_barrier()

        for t in range(rps // _CH):
            pltpu.sync_copy(s_agg.at[pl.ds(r0 + t * _CH, _CH)], rows_v)
            pltpu.sync_copy(rows_v, agg_hbm.at[cid, pl.ds(r0 + t * _CH, _CH)])

        @pl.when(cid == 0)
        def _():
            pltpu.sync_copy(hist_v, cnt_hbm.at[pl.ds(sid * nb, nb)])

    return k(ue, dst, z_nd, jnp.zeros((nb,), jnp.float32))


def kernel(node_feats, edge_feats, edge_index,
           edge_W1, edge_b1, edge_g1, edge_be1,
           edge_W2, edge_b2, edge_g2, edge_be2,
           node_W1, node_b1, node_g1, node_be1,
           node_W2, node_b2, node_g2, node_be2):
    n, d = node_feats.shape
    e = edge_feats.shape[0]
    src = edge_index[0]
    dst = edge_index[1]

    row = lambda v: v.reshape(1, d)

    ps, pd = _tc_proj(node_feats, edge_W1[0:d], edge_W1[d:2 * d])
    gs, gd = _sc_gather(ps, pd, src, dst)
    h1, s1, q1 = _edge_grid_call(
        _passa_body, e, d, [gs, gd, edge_feats],
        [edge_W1[2 * d:3 * d], row(edge_b1)], 2, out_dtype=jnp.bfloat16)
    h2, s2, q2 = _edge_grid_call(
        functools.partial(_passb_body, n_rows=float(e)), e, d, [h1],
        [s1, q1, row(edge_g1), row(edge_be1), edge_W2, row(edge_b2)], 2,
        out_dtype=jnp.bfloat16)
    (ue,) = _edge_grid_call(
        functools.partial(_passc_body, n_rows=float(e)), e, d,
        [h2, edge_feats], [s2, q2, row(edge_g2), row(edge_be2)], 0)

    half = -(-n // (_NC * _NS * 8)) * (_NS * 8)
    z_nd = jnp.zeros((_CH, d), jnp.float32)
    agg, cnt1d = _sc_scatter(ue, dst, z_nd, half)
    cnt = cnt1d.reshape(_NS, _NC * half).T

    un = _tc_node(node_feats, agg, cnt,
                  node_W1[0:d], node_W1[d:2 * d], row(node_b1),
                  row(node_g1), row(node_be1),
                  node_W2, row(node_b2), row(node_g2), row(node_be2))
    return (un, ue)


# speedup vs baseline: 2.9917x; 1.0125x over previous
import functools

import jax
import jax.numpy as jnp
from jax import lax
from jax.experimental import pallas as pl
from jax.experimental.pallas import tpu as pltpu
from jax.experimental.pallas import tpu_sc as plsc

_EPS = 1e-5
_NC = 2
_NS = 16
_NW = _NC * _NS
_CH = 80
_BLK = 2560
_CW = 16


def _proj_body(nf_ref, ws_ref, wd_ref, ps_ref, pd_ref):
    nf = nf_ref[...]
    ps_ref[...] = jnp.dot(nf, ws_ref[...], preferred_element_type=jnp.float32)
    pd_ref[...] = jnp.dot(nf, wd_ref[...], preferred_element_type=jnp.float32)


def _tc_proj(nf, ws, wd):
    n, d = nf.shape
    return pl.pallas_call(
        _proj_body,
        out_shape=(jax.ShapeDtypeStruct((n, d), jnp.float32),
                   jax.ShapeDtypeStruct((n, d), jnp.float32)),
    )(nf, ws, wd)


def _accum_stats(h, s_ref, q_ref):
    ps = jnp.sum(h, axis=0, keepdims=True)
    pq = jnp.sum(h * h, axis=0, keepdims=True)

    @pl.when(pl.program_id(0) == 0)
    def _():
        s_ref[...] = ps
        q_ref[...] = pq

    @pl.when(pl.program_id(0) != 0)
    def _():
        s_ref[...] += ps
        q_ref[...] += pq


def _passa_body(gs_ref, gd_ref, ef_ref, w_ref, b_ref, h1_ref, s_ref, q_ref):
    h = jnp.dot(ef_ref[...], w_ref[...], preferred_element_type=jnp.float32)
    h = h + gs_ref[...] + gd_ref[...] + b_ref[...]
    h1_ref[...] = h.astype(h1_ref.dtype)
    _accum_stats(h, s_ref, q_ref)


def _bn_affine(s, q, g, be, n):
    mu = s * (1.0 / n)
    var = q * (1.0 / n) - mu * mu
    inv = lax.rsqrt(var + _EPS) * g
    return inv, be - mu * inv


def _silu(x):
    return x * jax.nn.sigmoid(x)


def _passb_body(h1_ref, s1_ref, q1_ref, g1_ref, be1_ref, w2_ref, b2_ref,
                h2_ref, s_ref, q_ref, *, n_rows):
    scale, shift = _bn_affine(s1_ref[...], q1_ref[...], g1_ref[...],
                              be1_ref[...], n_rows)
    a = _silu(h1_ref[...].astype(jnp.float32) * scale + shift)
    h2 = jnp.dot(a, w2_ref[...], preferred_element_type=jnp.float32)
    h2 = h2 + b2_ref[...]
    h2_ref[...] = h2.astype(h2_ref.dtype)
    _accum_stats(h2, s_ref, q_ref)


def _passc_body(h2_ref, ef_ref, s2_ref, q2_ref, g2_ref, be2_ref, ue_ref,
                *, n_rows):
    scale, shift = _bn_affine(s2_ref[...], q2_ref[...], g2_ref[...],
                              be2_ref[...], n_rows)
    ue_ref[...] = _silu(h2_ref[...].astype(jnp.float32) * scale + shift
                        + ef_ref[...])


def _edge_grid_call(body, e, d, row_ins, small_ins, n_small_out,
                    out_dtype=jnp.float32):
    g = e // _BLK
    bs_rows = pl.BlockSpec((_BLK, d), lambda i: (i, 0))
    bs_vec = pl.BlockSpec((1, d), lambda i: (0, 0))
    bs_mat = pl.BlockSpec((d, d), lambda i: (0, 0))
    in_specs = [bs_rows] * len(row_ins)
    for x in small_ins:
        in_specs.append(bs_mat if x.shape[0] == d else bs_vec)
    out_shape = [jax.ShapeDtypeStruct((e, d), out_dtype)]
    out_specs = [bs_rows]
    for _ in range(n_small_out):
        out_shape.append(jax.ShapeDtypeStruct((1, d), jnp.float32))
        out_specs.append(bs_vec)
    return pl.pallas_call(
        body,
        grid=(g,),
        in_specs=in_specs,
        out_specs=out_specs,
        out_shape=out_shape,
    )(*row_ins, *small_ins)


def _node_body(nf_ref, agg_ref, cnt_ref,
               w1a_ref, w1b_ref, b1_ref, g1_ref, be1_ref,
               w2_ref, b2_ref, g2_ref, be2_ref, un_ref, *, n_rows):
    nf = nf_ref[...]
    n = nf.shape[0]
    deg = jnp.sum(cnt_ref[...], axis=1, keepdims=True)[0:n]
    agg = jnp.concatenate([agg_ref[0], agg_ref[1]], axis=0)[0:n]
    hg = agg / jnp.maximum(deg, 1.0)
    h = (jnp.dot(nf, w1a_ref[...], preferred_element_type=jnp.float32)
         + jnp.dot(hg, w1b_ref[...], preferred_element_type=jnp.float32)
         + b1_ref[...])
    mu = jnp.mean(h, axis=0, keepdims=True)
    var = jnp.mean(h * h, axis=0, keepdims=True) - mu * mu
    h = (h - mu) * lax.rsqrt(var + _EPS) * g1_ref[...] + be1_ref[...]
    h = _silu(h)
    h = jnp.dot(h, w2_ref[...], preferred_element_type=jnp.float32) + b2_ref[...]
    mu = jnp.mean(h, axis=0, keepdims=True)
    var = jnp.mean(h * h, axis=0, keepdims=True) - mu * mu
    h = (h - mu) * lax.rsqrt(var + _EPS) * g2_ref[...] + be2_ref[...]
    un_ref[...] = _silu(h + nf)


def _tc_node(nf, agg, cnt, w1a, w1b, b1, g1, be1, w2, b2, g2, be2):
    n, d = nf.shape
    return pl.pallas_call(
        functools.partial(_node_body, n_rows=float(n)),
        out_shape=jax.ShapeDtypeStruct((n, d), jnp.float32),
    )(nf, agg, cnt, w1a, w1b, b1, g1, be1, w2, b2, g2, be2)


def _sc_gather(ps, pd, src, dst):
    n, d = ps.shape
    e = src.shape[0]
    per_w = e // _NW
    nch = per_w // _CH
    mesh = plsc.VectorSubcoreMesh(core_axis_name="c", subcore_axis_name="s")

    nq = 4

    @functools.partial(
        pl.kernel,
        out_type=(jax.ShapeDtypeStruct((e, d), jnp.float32),
                  jax.ShapeDtypeStruct((e, d), jnp.float32)),
        mesh=mesh,
        scratch_types=(
            [pltpu.VMEM((_CH,), jnp.int32)] * (2 * nq)
            + [pltpu.VMEM((_CH, d), jnp.float32)] * (2 * nq)
            + [pltpu.SemaphoreType.DMA] * (3 * nq)
        ),
    )
    def k(ps_hbm, pd_hbm, src_hbm, dst_hbm, gs_hbm, gd_hbm, *bufs):
        idx_s = bufs[0:nq]
        idx_d = bufs[nq:2 * nq]
        row_s = bufs[2 * nq:3 * nq]
        row_d = bufs[3 * nq:4 * nq]
        sem_i = bufs[4 * nq:5 * nq]
        sem_g = bufs[5 * nq:6 * nq]
        sem_w = bufs[6 * nq:7 * nq]
        wid = lax.axis_index("s") * _NC + lax.axis_index("c")
        base = pl.multiple_of(wid * per_w, 8)

        def body(t, carry):
            offs = [pl.multiple_of(base + (nq * t + q) * _CH, 8)
                    for q in range(nq)]
            ii = [(pltpu.async_copy(src_hbm.at[pl.ds(offs[q], _CH)],
                                    idx_s[q], sem_i[q]),
                   pltpu.async_copy(dst_hbm.at[pl.ds(offs[q], _CH)],
                                    idx_d[q], sem_i[q]))
                  for q in range(nq)]
            gg = []
            for q in range(nq):
                ii[q][0].wait()
                ii[q][1].wait()
                gg.append((pltpu.async_copy(ps_hbm.at[idx_s[q]], row_s[q],
                                            sem_g[q]),
                           pltpu.async_copy(pd_hbm.at[idx_d[q]], row_d[q],
                                            sem_g[q])))
            ww = []
            for q in range(nq):
                gg[q][0].wait()
                gg[q][1].wait()
                ww.append((pltpu.async_copy(row_s[q],
                                            gs_hbm.at[pl.ds(offs[q], _CH)],
                                            sem_w[q]),
                           pltpu.async_copy(row_d[q],
                                            gd_hbm.at[pl.ds(offs[q], _CH)],
                                            sem_w[q])))
            for q in range(nq):
                ww[q][0].wait()
                ww[q][1].wait()
            return carry

        lax.fori_loop(0, nch // nq, body, 0)
        for r in range(nch - (nch // nq) * nq):
            off = pl.multiple_of(base + ((nch // nq) * nq + r) * _CH, 8)
            pltpu.sync_copy(src_hbm.at[pl.ds(off, _CH)], idx_s[0])
            pltpu.sync_copy(dst_hbm.at[pl.ds(off, _CH)], idx_d[0])
            cs = pltpu.async_copy(ps_hbm.at[idx_s[0]], row_s[0], sem_g[0])
            cd = pltpu.async_copy(pd_hbm.at[idx_d[0]], row_d[0], sem_g[1])
            cs.wait()
            cd.wait()
            pltpu.sync_copy(row_s[0], gs_hbm.at[pl.ds(off, _CH)])
            pltpu.sync_copy(row_d[0], gd_hbm.at[pl.ds(off, _CH)])

    return k(ps, pd, src, dst)


def _sc_scatter(ue, dst, z_nd, half):
    e, d = ue.shape
    per_s = e // _NS
    nch = per_s // _CH
    rps = half // _NS
    mesh = plsc.VectorSubcoreMesh(core_axis_name="c", subcore_axis_name="s")

    nb = _NC * half

    @functools.partial(
        pl.kernel,
        out_type=(jax.ShapeDtypeStruct((_NC, half, d), jnp.float32),
                  jax.ShapeDtypeStruct((_NS * nb,), jnp.float32)),
        mesh=mesh,
        compiler_params=pltpu.CompilerParams(needs_layout_passes=False),
        scratch_types=(
            [pltpu.VMEM((_CH,), jnp.int32)] * 3
            + [pltpu.VMEM((_CH, d), jnp.float32)] * 3
            + [pltpu.VMEM((nb,), jnp.float32)]
            + [pltpu.SemaphoreType.DMA] * 6
            + [pltpu.VMEM_SHARED((half + 8, d), jnp.float32)]
        ),
    )
    def k(ue_hbm, dst_hbm, znd_hbm, z1_hbm, agg_hbm, cnt_hbm,
          idx_v, idx_v1, idx_v2, rows_v, rows_v1, rows_v2, hist_v,
          sem_i0, sem_i1, sem_i2, sem_r0, sem_r1, sem_r2, s_agg):
        cid = lax.axis_index("c")
        sid = lax.axis_index("s")
        base = pl.multiple_of(sid * per_s, 8)
        r0 = pl.multiple_of(sid * rps, 8)
        lo = cid * half

        pltpu.sync_copy(znd_hbm, rows_v)
        for t in range(rps // _CH):
            pltpu.sync_copy(rows_v, s_agg.at[pl.ds(r0 + t * _CH, _CH)])
        pltpu.sync_copy(z1_hbm, hist_v)
        plsc.subcore_barrier()

        def hist_and_remap(ix):
            for j in range(_CH // 16):
                g = ix[pl.ds(j * 16, 16)]

                @pl.when(cid == 0)
                def _():
                    plsc.addupdate_scatter(hist_v, [g],
                                           jnp.ones((16,), jnp.float32))
                v = g - lo
                ok = (v >= 0) & (v < half)
                ix[pl.ds(j * 16, 16)] = jnp.where(ok, v, half)

        ixs = [idx_v, idx_v1, idx_v2]
        rws = [rows_v, rows_v1, rows_v2]
        sis = [sem_i0, sem_i1, sem_i2]
        srs = [sem_r0, sem_r1, sem_r2]

        def body(t, carry):
            offs = [pl.multiple_of(base + (3 * t + q) * _CH, 8)
                    for q in range(3)]
            cps = [(pltpu.async_copy(dst_hbm.at[pl.ds(offs[q], _CH)],
                                     ixs[q], sis[q]),
                    pltpu.async_copy(ue_hbm.at[pl.ds(offs[q], _CH)],
                                     rws[q], srs[q]))
                   for q in range(3)]
            for q in range(3):
                cps[q][0].wait()
                hist_and_remap(ixs[q])
                cps[q][1].wait()
                pltpu.sync_copy(rws[q], s_agg.at[ixs[q]], add=True)
            return carry

        lax.fori_loop(0, nch // 3, body, 0)
        for r in range(nch - (nch // 3) * 3):
            off = pl.multiple_of(base + ((nch // 3) * 3 + r) * _CH, 8)
            pltpu.sync_copy(dst_hbm.at[pl.ds(off, _CH)], idx_v)
            pltpu.sync_copy(ue_hbm.at[pl.ds(off, _CH)], rows_v)
            hist_and_remap(idx_v)
            pltpu.sync_copy(rows_v, s_agg.at[idx_v], add=True)
        plsc.subcore_barrier()

        for t in range(rps // _CH):
            pltpu.sync_copy(s_agg.at[pl.ds(r0 + t * _CH, _CH)], rows_v)
            pltpu.sync_copy(rows_v, agg_hbm.at[cid, pl.ds(r0 + t * _CH, _CH)])

        @pl.when(cid == 0)
        def _():
            pltpu.sync_copy(hist_v, cnt_hbm.at[pl.ds(sid * nb, nb)])

    return k(ue, dst, z_nd, jnp.zeros((nb,), jnp.float32))


def kernel(node_feats, edge_feats, edge_index,
           edge_W1, edge_b1, edge_g1, edge_be1,
           edge_W2, edge_b2, edge_g2, edge_be2,
           node_W1, node_b1, node_g1, node_be1,
           node_W2, node_b2, node_g2, node_be2):
    n, d = node_feats.shape
    e = edge_feats.shape[0]
    src = edge_index[0]
    dst = edge_index[1]

    row = lambda v: v.reshape(1, d)

    ps, pd = _tc_proj(node_feats, edge_W1[0:d], edge_W1[d:2 * d])
    gs, gd = _sc_gather(ps, pd, src, dst)
    h1, s1, q1 = _edge_grid_call(
        _passa_body, e, d, [gs, gd, edge_feats],
        [edge_W1[2 * d:3 * d], row(edge_b1)], 2, out_dtype=jnp.bfloat16)
    h2, s2, q2 = _edge_grid_call(
        functools.partial(_passb_body, n_rows=float(e)), e, d, [h1],
        [s1, q1, row(edge_g1), row(edge_be1), edge_W2, row(edge_b2)], 2,
        out_dtype=jnp.bfloat16)
    (ue,) = _edge_grid_call(
        functools.partial(_passc_body, n_rows=float(e)), e, d,
        [h2, edge_feats], [s2, q2, row(edge_g2), row(edge_be2)], 0)

    half = -(-n // (_NC * _NS * 8)) * (_NS * 8)
    z_nd = jnp.zeros((_CH, d), jnp.float32)
    agg, cnt1d = _sc_scatter(ue, dst, z_nd, half)
    cnt = cnt1d.reshape(_NS, _NC * half).T

    un = _tc_node(node_feats, agg, cnt,
                  node_W1[0:d], node_W1[d:2 * d], row(node_b1),
                  row(node_g1), row(node_be1),
                  node_W2, row(node_b2), row(node_g2), row(node_be2))
    return (un, ue)
